# Initial kernel scaffold; baseline (speedup 1.0000x reference)
import jax
import jax.numpy as jnp
from jax.experimental import pallas as pl


def kernel(vertices, faces, face_edges, params):
    raise NotImplementedError("write your pallas kernel here")


# speedup vs baseline: 11.2223x; 11.2223x over previous
import functools

import jax
import jax.numpy as jnp
import numpy as np
from jax import lax
from jax.experimental import pallas as pl
from jax.experimental.pallas import tpu as pltpu
from jax.experimental.pallas import tpu_sc as plsc

NV, NF, E = 5000, 10000, 320000
DIM_CODEBOOK = 192
FACE_DIM = 208
ENC_DIMS = (64, 128, 256)

NC, NS = 2, 16
CHUNK = 80


def _round_up(x, m):
    return (x + m - 1) // m * m


@functools.partial(jax.jit, static_argnames=("n_rows", "m_pad", "w", "nct", "identity_src"))
def _sc_scatter_add(t0, t1, src2, dst2, zeros, *, n_rows, m_pad, w, nct, identity_src):
    mesh = plsc.VectorSubcoreMesh(
        core_axis_name="c", subcore_axis_name="s", num_cores=NC, num_subcores=NS)
    rpt = m_pad // NS
    src2 = src2.reshape(NS, nct, CHUNK)
    dst2 = dst2.reshape(NS, nct, CHUNK)
    zeros = zeros.reshape(NS, rpt, w)

    def body(t0_ref, t1_ref, src_ref, dst_ref, zeros_ref, out0_ref, out1_ref,
             src_v, dst_v, rows_v, acc, sem):
        c = lax.axis_index("c")
        s = lax.axis_index("s")
        r0 = s * rpt
        pltpu.sync_copy(zeros_ref.at[s], acc.at[pl.ds(r0, rpt)])
        if not identity_src:
            pltpu.sync_copy(src_ref.at[s], src_v)
        pltpu.sync_copy(dst_ref.at[s], dst_v)
        plsc.subcore_barrier()

        def step(ci, carry):
            if identity_src:
                row0 = (s * nct + ci) * CHUNK

                @pl.when(c == 0)
                def _():
                    pltpu.sync_copy(t0_ref.at[pl.ds(row0, CHUNK)], rows_v)

                @pl.when(c == 1)
                def _():
                    pltpu.sync_copy(t1_ref.at[pl.ds(row0, CHUNK)], rows_v)
            else:
                @pl.when(c == 0)
                def _():
                    pltpu.async_copy(t0_ref.at[src_v.at[ci]], rows_v, sem).wait()

                @pl.when(c == 1)
                def _():
                    pltpu.async_copy(t1_ref.at[src_v.at[ci]], rows_v, sem).wait()
            pltpu.sync_copy(rows_v, acc.at[dst_v.at[ci]], add=True)
            return carry

        lax.fori_loop(0, nct, step, 0)
        plsc.subcore_barrier()

        @pl.when(c == 0)
        def _():
            pltpu.sync_copy(acc.at[pl.ds(r0, rpt)], out0_ref.at[s])

        @pl.when(c == 1)
        def _():
            pltpu.sync_copy(acc.at[pl.ds(r0, rpt)], out1_ref.at[s])

    scratch = [
        pltpu.VMEM((max(nct, 1), CHUNK), jnp.int32),
        pltpu.VMEM((nct, CHUNK), jnp.int32),
        pltpu.VMEM((CHUNK, w), jnp.float32),
        pltpu.VMEM_SHARED((m_pad, w), jnp.float32),
        pltpu.SemaphoreType.DMA,
    ]
    out_type = (jax.ShapeDtypeStruct((NS, rpt, w), jnp.float32),
                jax.ShapeDtypeStruct((NS, rpt, w), jnp.float32))
    o0, o1 = pl.kernel(
        body, out_type=out_type, mesh=mesh, scratch_types=scratch,
        compiler_params=pltpu.CompilerParams(use_tc_tiling_on_sc=False))(
        t0, t1, src2, dst2, zeros)
    return o0.reshape(m_pad, w), o1.reshape(m_pad, w)


def _tc_call(body, n_rows, bm, in_specs_widths, out_w, *args):
    grid = (n_rows // bm,)
    in_specs = []
    for a, kind in zip(args, in_specs_widths):
        if kind == "row":
            in_specs.append(pl.BlockSpec((bm, a.shape[1]), lambda i: (i, 0)))
        else:
            in_specs.append(pl.BlockSpec(a.shape, lambda i: (0,) * a.ndim))
    return pl.pallas_call(
        body,
        grid=grid,
        in_specs=in_specs,
        out_specs=pl.BlockSpec((bm, out_w), lambda i: (i, 0)),
        out_shape=jax.ShapeDtypeStruct((n_rows, out_w), jnp.float32),
    )(*args)


def _aug_proj(x, Wp, bp, D, w2):
    bm = 400

    def body(x_ref, w_ref, b_ref, o_ref):
        xp = jnp.maximum(
            jnp.dot(x_ref[...], w_ref[...], preferred_element_type=jnp.float32)
            + b_ref[...], 0.0)
        o_ref[:, :D] = xp
        o_ref[:, D:D + 16] = jnp.ones((bm, 16), jnp.float32)
        if w2 > D + 16:
            o_ref[:, D + 16:] = jnp.zeros((bm, w2 - D - 16), jnp.float32)

    return _tc_call(body, NF, bm, ("row", "full", "full"), w2, x, Wp, bp)


def _sage_out(m0, m1, aug, Wl, Wr, b, D, w, dout):
    bm = 400

    def body(m0_ref, m1_ref, aug_ref, wl_ref, wr_ref, b_ref, o_ref):
        m0v = m0_ref[...]
        m1v = m1_ref[...]
        msg = jnp.concatenate([m0v, m1v[:, :D - w]], axis=1)
        cnt = m1v[:, D - w:D - w + 1]
        mean = msg / jnp.maximum(cnt, 1.0)
        xp = aug_ref[:, :D]
        out = (jnp.dot(mean, wl_ref[...], preferred_element_type=jnp.float32)
               + jnp.dot(xp, wr_ref[...], preferred_element_type=jnp.float32)
               + b_ref[...])
        nrm = jnp.sqrt(jnp.sum(out * out, axis=1, keepdims=True))
        out = out / jnp.maximum(nrm, 1e-12)
        o_ref[...] = out * jax.nn.sigmoid(out)

    return _tc_call(body, NF, bm, ("row", "row", "row", "full", "full", "full"),
                    dout, m0, m1, aug, Wl, Wr, b)


def _codes_aug(h, Wc, bc):
    bm = 400

    def body(h_ref, w_ref, b_ref, o_ref):
        codes = (jnp.dot(h_ref[...], w_ref[...], preferred_element_type=jnp.float32)
                 + b_ref[...])
        for k in range(3):
            o_ref[:, k * 224:k * 224 + 192] = codes[:, k * 192:(k + 1) * 192]
            o_ref[:, k * 224 + 192:k * 224 + 208] = jnp.ones((bm, 16), jnp.float32)
            o_ref[:, k * 224 + 208:(k + 1) * 224] = jnp.zeros((bm, 16), jnp.float32)

    return _tc_call(body, NF, bm, ("row", "full", "full"), 672, h, Wc, bc)


def _vert_div(m0, m1):
    bm = 200

    def body(m0_ref, m1_ref, o_ref):
        den = jnp.maximum(m1_ref[:, 80:81], 1e-5)
        o_ref[:, :112] = m0_ref[...] / den
        o_ref[:, 112:] = m1_ref[:, :80] / den

    return _tc_call(body, NV, bm, ("row", "row"), DIM_CODEBOOK, m0, m1)


def _l2norm(t, eps=1e-12):
    return t / jnp.clip(jnp.linalg.norm(t, axis=-1, keepdims=True), eps)


def _discretize(t, lo, hi, num=128):
    t = (t - lo) / (hi - lo) * num - 0.5
    return jnp.clip(jnp.round(t).astype(jnp.int32), 0, num - 1)


def _face_feats(vertices, faces, params):
    v = vertices[0]
    f = faces[0]
    fc = v[f]
    fc3 = jnp.pad(fc, ((0, 0), (0, 0), (0, 1)))
    shifted = jnp.concatenate([fc3[:, -1:], fc3[:, :-1]], axis=1)
    z = jnp.sum(_l2norm(fc3) * _l2norm(shifted), axis=-1)
    angles = jnp.arccos(jnp.clip(z, -1 + 1e-5, 1 - 1e-5))
    ev = fc3 - shifted
    cross = jnp.cross(ev[:, 0], ev[:, 1])
    normals = _l2norm(cross)
    area = jnp.linalg.norm(cross, axis=-1, keepdims=True) * 0.5
    ce = params['coor_embed'][_discretize(fc, -1.0, 1.0)].reshape(NF, -1)
    ae = params['angle_embed'][_discretize(angles, 0.0, float(np.pi))].reshape(NF, -1)
    re = params['area_embed'][_discretize(area, 0.0, 4.0)].reshape(NF, -1)
    ne = params['normal_embed'][_discretize(normals, -1.0, 1.0)].reshape(NF, -1)
    return jnp.concatenate([ce, ae, re, ne], axis=-1)


def kernel(vertices, faces, face_edges, params):
    x = _face_feats(vertices, faces, params)

    src = face_edges[0, :, 0].reshape(E // CHUNK, CHUNK)
    dst = face_edges[0, :, 1].reshape(E // CHUNK, CHUNK)
    m_pad_f = _round_up(NF + 8, NS * 8)

    dims = [FACE_DIM] + list(ENC_DIMS)
    h = x
    for i, p in enumerate(params['sage']):
        D = dims[i]
        w = _round_up(D + 16, 32) // 2
        aug = _aug_proj(h, p['Wp'], p['bp'].reshape(1, -1), D, 2 * w)
        t0 = aug[:, :w]
        t1 = aug[:, w:]
        zeros = jnp.zeros((m_pad_f, w), jnp.float32)
        m0, m1 = _sc_scatter_add(
            t0, t1, src, dst, zeros,
            n_rows=NF, m_pad=m_pad_f, w=w, nct=(E // CHUNK) // NS,
            identity_src=False)
        h = _sage_out(m0[:NF], m1[:NF], aug, p['Wl'], p['Wr'],
                      p['b'].reshape(1, -1), D, w, dims[i + 1])

    caug = _codes_aug(h, params['Wc'], params['bc'].reshape(1, -1))
    tbl = caug.reshape(3 * NF, 224)
    ep = _round_up(3 * NF, NS * CHUNK)
    tbl = jnp.pad(tbl, ((0, ep - 3 * NF), (0, 0)))
    fdst = jnp.pad(faces[0].reshape(-1), (0, ep - 3 * NF),
                   constant_values=NV).reshape(ep // CHUNK, CHUNK)
    m_pad_v = _round_up(NV + 8, NS * 8)
    zeros_v = jnp.zeros((m_pad_v, 112), jnp.float32)
    vm0, vm1 = _sc_scatter_add(
        tbl[:, :112], tbl[:, 112:], fdst, fdst, zeros_v,
        n_rows=ep, m_pad=m_pad_v, w=112, nct=(ep // CHUNK) // NS,
        identity_src=True)
    out = _vert_div(vm0[:NV], vm1[:NV])
    return out.reshape(1, NV, DIM_CODEBOOK)
